# Initial kernel scaffold; baseline (speedup 1.0000x reference)
import jax
import jax.numpy as jnp
from jax.experimental import pallas as pl


def kernel(x, W):
    raise NotImplementedError("write your pallas kernel here")


# speedup vs baseline: 1.9417x; 1.9417x over previous
import jax
import jax.numpy as jnp
from jax.experimental import pallas as pl
from jax.experimental.pallas import tpu as pltpu

_DIM = 2048
_DC = 1024
_NB = 512


def _pconv_block(x_ref, w_ref, o_ref):
    x1 = x_ref[0, :_DC, :]
    y1 = jax.lax.dot(
        w_ref[...].astype(jnp.bfloat16),
        x1.astype(jnp.bfloat16),
        preferred_element_type=jnp.float32,
    )
    o_ref[0, :_DC, :] = y1
    o_ref[0, _DC:, :] = x_ref[0, _DC:, :]


def kernel(x, W):
    b, dim, n = x.shape
    grid = (b, n // _NB)
    return pl.pallas_call(
        _pconv_block,
        grid=grid,
        in_specs=[
            pl.BlockSpec((1, _DIM, _NB), lambda i, j: (i, 0, j)),
            pl.BlockSpec((_DC, _DC), lambda i, j: (0, 0)),
        ],
        out_specs=pl.BlockSpec((1, _DIM, _NB), lambda i, j: (i, 0, j)),
        out_shape=jax.ShapeDtypeStruct((b, dim, n), x.dtype),
        compiler_params=pltpu.CompilerParams(
            dimension_semantics=("parallel", "parallel"),
        ),
    )(x, W)


# speedup vs baseline: 2.0150x; 1.0378x over previous
import jax
import jax.numpy as jnp
from jax.experimental import pallas as pl
from jax.experimental.pallas import tpu as pltpu

_DIM = 2048
_DC = 1024
_NB = 1024


def _pconv_block(x_ref, w_ref, o_ref):
    x1 = x_ref[0, :_DC, :]
    y1 = jax.lax.dot(
        w_ref[...].astype(jnp.bfloat16),
        x1.astype(jnp.bfloat16),
        preferred_element_type=jnp.float32,
    )
    o_ref[0, :_DC, :] = y1
    o_ref[0, _DC:, :] = x_ref[0, _DC:, :]


def kernel(x, W):
    b, dim, n = x.shape
    grid = (b, n // _NB)
    return pl.pallas_call(
        _pconv_block,
        grid=grid,
        in_specs=[
            pl.BlockSpec((1, _DIM, _NB), lambda i, j: (i, 0, j)),
            pl.BlockSpec((_DC, _DC), lambda i, j: (0, 0)),
        ],
        out_specs=pl.BlockSpec((1, _DIM, _NB), lambda i, j: (i, 0, j)),
        out_shape=jax.ShapeDtypeStruct((b, dim, n), x.dtype),
        compiler_params=pltpu.CompilerParams(
            dimension_semantics=("parallel", "parallel"),
        ),
    )(x, W)
